# Initial kernel scaffold; baseline (speedup 1.0000x reference)
import jax
import jax.numpy as jnp
from jax.experimental import pallas as pl


def kernel(i, encoding):
    raise NotImplementedError("write your pallas kernel here")


# speedup vs baseline: 1.4799x; 1.4799x over previous
import functools

import jax
import jax.numpy as jnp
from jax import lax
from jax.experimental import pallas as pl
from jax.experimental.pallas import tpu as pltpu
from jax.experimental.pallas import tpu_sc as plsc


@functools.lru_cache(maxsize=None)
def _make_gather(V, D, B):
    info = plsc.get_sparse_core_info()
    NC, NS = info.num_cores, info.num_subcores
    NW = NC * NS
    assert B % NW == 0
    b_per_w = B // NW
    CH = 32
    assert b_per_w % CH == 0
    n_chunks = b_per_w // CH
    mesh = plsc.VectorSubcoreMesh(core_axis_name="c", subcore_axis_name="s")

    @functools.partial(
        pl.kernel,
        mesh=mesh,
        out_type=jax.ShapeDtypeStruct((B, D), jnp.float32),
        scratch_types=[
            pltpu.VMEM((b_per_w,), jnp.int32),
            pltpu.VMEM((CH, D), jnp.float32),
            pltpu.SemaphoreType.DMA,
        ],
    )
    def k(idx_hbm, table_hbm, out_hbm, idx_v, rows_v, sem):
        wid = lax.axis_index("s") * NC + lax.axis_index("c")
        base = wid * b_per_w
        pltpu.sync_copy(idx_hbm.at[pl.ds(base, b_per_w)], idx_v)

        def body(c, carry):
            off = pl.multiple_of(c * CH, 8)
            idx_slice = idx_v.at[pl.ds(off, CH)]
            pltpu.async_copy(table_hbm.at[idx_slice], rows_v, sem).wait()
            pltpu.sync_copy(rows_v, out_hbm.at[pl.ds(base + off, CH)])
            return carry

        lax.fori_loop(0, n_chunks, body, 0)

    return k


def kernel(i, encoding):
    s, b = i.shape
    V, D = encoding.shape
    flat = i.reshape(-1).astype(jnp.int32)
    out = _make_gather(V, D, s * b)(flat, encoding)
    return out.reshape(s, b, D)


# speedup vs baseline: 1.5765x; 1.0653x over previous
import functools

import jax
import jax.numpy as jnp
from jax import lax
from jax.experimental import pallas as pl
from jax.experimental.pallas import tpu as pltpu
from jax.experimental.pallas import tpu_sc as plsc


@functools.lru_cache(maxsize=None)
def _make_gather(V, D, B):
    info = plsc.get_sparse_core_info()
    NC, NS = info.num_cores, info.num_subcores
    NW = NC * NS
    assert B % NW == 0
    b_per_w = B // NW
    CH = 32
    assert b_per_w % (2 * CH) == 0
    n_pairs = b_per_w // (2 * CH)
    mesh = plsc.VectorSubcoreMesh(core_axis_name="c", subcore_axis_name="s")

    @functools.partial(
        pl.kernel,
        mesh=mesh,
        out_type=jax.ShapeDtypeStruct((B, D), jnp.float32),
        scratch_types=[
            pltpu.VMEM((b_per_w,), jnp.int32),
            pltpu.VMEM((CH, D), jnp.float32),
            pltpu.VMEM((CH, D), jnp.float32),
            pltpu.SemaphoreType.DMA,
            pltpu.SemaphoreType.DMA,
            pltpu.SemaphoreType.DMA,
            pltpu.SemaphoreType.DMA,
        ],
    )
    def k(idx_hbm, table_hbm, out_hbm, idx_v, rows0, rows1, gs0, gs1, ss0, ss1):
        wid = lax.axis_index("s") * NC + lax.axis_index("c")
        base = wid * b_per_w
        pltpu.sync_copy(idx_hbm.at[pl.ds(base, b_per_w)], idx_v)

        def start_g(c, buf, sem):
            off = pl.multiple_of(c * CH, 8)
            pltpu.async_copy(table_hbm.at[idx_v.at[pl.ds(off, CH)]], buf, sem)

        def wait_g(buf, sem):
            pltpu.make_async_copy(
                table_hbm.at[idx_v.at[pl.ds(0, CH)]], buf, sem
            ).wait()

        def start_s(c, buf, sem):
            off = pl.multiple_of(c * CH, 8)
            pltpu.async_copy(buf, out_hbm.at[pl.ds(base + off, CH)], sem)

        def wait_s(buf, sem):
            pltpu.make_async_copy(buf, out_hbm.at[pl.ds(base, CH)], sem).wait()

        start_g(0, rows0, gs0)

        def body(g, carry):
            c0 = g * 2
            c1 = c0 + 1
            wait_g(rows0, gs0)
            start_s(c0, rows0, ss0)
            pl.when(g > 0)(lambda: wait_s(rows1, ss1))
            start_g(c1, rows1, gs1)
            wait_g(rows1, gs1)
            start_s(c1, rows1, ss1)
            wait_s(rows0, ss0)
            pl.when(g + 1 < n_pairs)(lambda: start_g(c0 + 2, rows0, gs0))
            return carry

        lax.fori_loop(0, n_pairs, body, 0)
        wait_s(rows1, ss1)

    return k


def kernel(i, encoding):
    s, b = i.shape
    V, D = encoding.shape
    flat = i.reshape(-1).astype(jnp.int32)
    out = _make_gather(V, D, s * b)(flat, encoding)
    return out.reshape(s, b, D)


# speedup vs baseline: 3.2288x; 2.0481x over previous
import functools

import jax
import jax.numpy as jnp
from jax import lax
from jax.experimental import pallas as pl
from jax.experimental.pallas import tpu as pltpu
from jax.experimental.pallas import tpu_sc as plsc


@functools.lru_cache(maxsize=None)
def _make_gather(V, D, S, B4):
    B = S * B4
    info = plsc.get_sparse_core_info()
    NC, NS = info.num_cores, info.num_subcores
    NW = NC * NS
    assert B % NW == 0
    b_per_w = B // NW
    CH = 32
    assert b_per_w % (2 * CH) == 0
    n_pairs = b_per_w // (2 * CH)
    mesh = plsc.VectorSubcoreMesh(core_axis_name="c", subcore_axis_name="s")

    @functools.partial(
        pl.kernel,
        mesh=mesh,
        out_type=jax.ShapeDtypeStruct((S, B4, D), jnp.float32),
        scratch_types=[
            pltpu.VMEM((b_per_w,), jnp.int32),
            pltpu.VMEM((CH, D), jnp.float32),
            pltpu.VMEM((CH, D), jnp.float32),
            pltpu.SemaphoreType.DMA,
            pltpu.SemaphoreType.DMA,
            pltpu.SemaphoreType.DMA,
            pltpu.SemaphoreType.DMA,
        ],
    )
    def k(idx_hbm, table_hbm, out3d, idx_v, rows0, rows1, gs0, gs1, ss0, ss1):
        out_hbm = out3d.reshape(B, D)
        wid = lax.axis_index("s") * NC + lax.axis_index("c")
        base = wid * b_per_w
        pltpu.sync_copy(idx_hbm.at[pl.ds(base, b_per_w)], idx_v)

        def start_g(c, buf, sem):
            off = pl.multiple_of(c * CH, 8)
            pltpu.async_copy(table_hbm.at[idx_v.at[pl.ds(off, CH)]], buf, sem)

        def wait_g(buf, sem):
            pltpu.make_async_copy(
                table_hbm.at[idx_v.at[pl.ds(0, CH)]], buf, sem
            ).wait()

        def start_s(c, buf, sem):
            off = pl.multiple_of(c * CH, 8)
            pltpu.async_copy(buf, out_hbm.at[pl.ds(base + off, CH)], sem)

        def wait_s(buf, sem):
            pltpu.make_async_copy(buf, out_hbm.at[pl.ds(base, CH)], sem).wait()

        start_g(0, rows0, gs0)

        def body(g, carry):
            c0 = g * 2
            c1 = c0 + 1
            wait_g(rows0, gs0)
            start_s(c0, rows0, ss0)
            pl.when(g > 0)(lambda: wait_s(rows1, ss1))
            start_g(c1, rows1, gs1)
            wait_g(rows1, gs1)
            start_s(c1, rows1, ss1)
            wait_s(rows0, ss0)
            pl.when(g + 1 < n_pairs)(lambda: start_g(c0 + 2, rows0, gs0))
            return carry

        lax.fori_loop(0, n_pairs, body, 0)
        wait_s(rows1, ss1)

    return k


def kernel(i, encoding):
    s, b = i.shape
    V, D = encoding.shape
    flat = i.reshape(-1).astype(jnp.int32)
    return _make_gather(V, D, s, b)(flat, encoding)


# speedup vs baseline: 3.5406x; 1.0966x over previous
import functools

import jax
import jax.numpy as jnp
from jax import lax
from jax.experimental import pallas as pl
from jax.experimental.pallas import tpu as pltpu
from jax.experimental.pallas import tpu_sc as plsc


@functools.lru_cache(maxsize=None)
def _make_gather(V, D, S, B4):
    B = S * B4
    info = plsc.get_sparse_core_info()
    NC, NS = info.num_cores, info.num_subcores
    NW = NC * NS
    assert B % NW == 0
    b_per_w = B // NW
    CH = 16
    NBUF = 4
    assert b_per_w % (NBUF * CH) == 0
    n_chunks = b_per_w // CH
    n_groups = n_chunks // NBUF
    mesh = plsc.VectorSubcoreMesh(core_axis_name="c", subcore_axis_name="s")

    @functools.partial(
        pl.kernel,
        mesh=mesh,
        out_type=jax.ShapeDtypeStruct((S, B4, D), jnp.float32),
        scratch_types=[
            pltpu.VMEM((b_per_w,), jnp.int32),
        ]
        + [pltpu.VMEM((CH, D), jnp.float32)] * NBUF
        + [pltpu.SemaphoreType.DMA] * (2 * NBUF),
    )
    def k(idx_hbm, table_hbm, out3d, idx_v, *bufsem):
        bufs = bufsem[:NBUF]
        gs = bufsem[NBUF : 2 * NBUF]
        ss = bufsem[2 * NBUF :]
        out_hbm = out3d.reshape(B, D)
        wid = lax.axis_index("s") * NC + lax.axis_index("c")
        base = wid * b_per_w
        pltpu.sync_copy(idx_hbm.at[pl.ds(base, b_per_w)], idx_v)

        def start_g(c, j):
            off = pl.multiple_of(c * CH, 8)
            pltpu.async_copy(table_hbm.at[idx_v.at[pl.ds(off, CH)]], bufs[j], gs[j])

        def wait_g(j):
            pltpu.make_async_copy(
                table_hbm.at[idx_v.at[pl.ds(0, CH)]], bufs[j], gs[j]
            ).wait()

        def start_s(c, j):
            off = pl.multiple_of(c * CH, 8)
            pltpu.async_copy(bufs[j], out_hbm.at[pl.ds(base + off, CH)], ss[j])

        def wait_s(j):
            pltpu.make_async_copy(bufs[j], out_hbm.at[pl.ds(base, CH)], ss[j]).wait()

        start_g(0, 0)
        start_g(1, 1)

        def body(g, carry):
            for j in range(NBUF):
                c = g * NBUF + j
                wait_g(j)
                start_s(c, j)
                jn = (j + 2) % NBUF
                pl.when(c >= 2)(lambda: wait_s(jn))
                pl.when(c + 2 < n_chunks)(lambda: start_g(c + 2, jn))
            return carry

        lax.fori_loop(0, n_groups, body, 0)
        wait_s((n_chunks - 2) % NBUF)
        wait_s((n_chunks - 1) % NBUF)

    return k


def kernel(i, encoding):
    s, b = i.shape
    V, D = encoding.shape
    flat = i.reshape(-1).astype(jnp.int32)
    return _make_gather(V, D, s, b)(flat, encoding)
